# baseline (speedup 1.0000x reference)
import jax
import jax.numpy as jnp
from jax.experimental import pallas as pl
from jax.experimental.pallas import tpu as pltpu
from jax.experimental.pallas import tpu_sc as plsc

B, N, D, C, K = 4, 4096, 1024, 2, 256

NB_A = 1024
NB_SEG = 2048


def _router_body(z_ref, W1_ref, b1_ref, W2_ref, b2_ref, scores_ref):
    zb = z_ref[...]
    h = jnp.maximum(
        jax.lax.dot_general(zb, W1_ref[...], (((1,), (0,)), ((), ())),
                            preferred_element_type=jnp.float32) + b1_ref[...],
        0.0)
    s = jax.lax.dot_general(h, W2_ref[...], (((1,), (0,)), ((), ())),
                            preferred_element_type=jnp.float32) + b2_ref[...]
    scores_ref[...] = s


def _seg_body(z_ref, Wh_ref, bh_ref, seg_ref):
    seg_ref[...] = jax.lax.dot_general(
        z_ref[...], Wh_ref[...], (((1,), (0,)), ((), ())),
        preferred_element_type=jnp.float32) + bh_ref[...]


def _clip_body(w_ref, seg_ref, clip_ref):
    rows = []
    for b in range(B):
        rows.append(jax.lax.dot_general(
            w_ref[b:b + 1, :], seg_ref[b * N:(b + 1) * N, :],
            (((1,), (0,)), ((), ())), preferred_element_type=jnp.float32))
    clip_ref[...] = jnp.concatenate(rows, axis=0)


def _sortable_i32(bits):
    return jnp.where(bits < 0, bits ^ jnp.int32(0x7FFFFFFF), bits)


_NV = N // 16


def _thresh_body(scores_ref, tn_ref):
    kk = _sortable_i32(jax.lax.bitcast_convert_type(scores_ref[...], jnp.int32))
    msb = jnp.int32(-2147483648)

    def step(i, prefix_u):
        cand_u = prefix_u | (jnp.int32(1) << (jnp.int32(31) - i))
        cand_s = cand_u ^ msb
        cnt = jnp.sum((kk >= cand_s).astype(jnp.int32), axis=1, keepdims=True)
        return jnp.where(cnt >= K, cand_u, prefix_u)

    prefix_u = jax.lax.fori_loop(0, 32, step, jnp.zeros((B, 1), jnp.int32))
    t_s = prefix_u ^ msb
    n_gt = jnp.sum((kk > t_s).astype(jnp.int32), axis=1, keepdims=True)
    need = K - n_gt
    tn_ref[...] = jnp.concatenate(
        [jnp.broadcast_to(t_s, (B, 16)), jnp.broadcast_to(need, (B, 16))],
        axis=1)


def _sc_apply_body(scores_hbm, tn_hbm, w_hbm, s_v, tn_v, w_v):
    cid = jax.lax.axis_index("c")
    sid = jax.lax.axis_index("s")
    wid = sid * 2 + cid

    @pl.when(wid < B)
    def _():
        b = wid
        pltpu.sync_copy(scores_hbm.at[pl.ds(pl.multiple_of(b * N, 8), N)], s_v)
        pltpu.sync_copy(tn_hbm.at[pl.ds(pl.multiple_of(b * 32, 8), 32)], tn_v)

        t_vec = tn_v[pl.ds(0, 16)]
        need_v = tn_v[pl.ds(16, 16)]
        inv_k = jnp.float32(1.0 / K)

        def fin(i, ceq_v):
            sv = s_v[pl.ds(i * 16, 16)]
            bits = jax.lax.bitcast_convert_type(sv, jnp.int32)
            ks = jnp.where(bits < 0, bits ^ jnp.int32(0x7FFFFFFF), bits)
            gt = ks > t_vec
            eq = ks == t_vec
            ec = plsc.cumsum(eq.astype(jnp.int32))
            sel = jnp.logical_or(
                gt, jnp.logical_and(eq, (ceq_v + ec) <= need_v))
            w_v[pl.ds(i * 16, 16)] = jnp.where(sel, inv_k, 0.0)
            return ceq_v + plsc.all_reduce_population_count(eq)
        jax.lax.fori_loop(0, _NV, fin, jnp.zeros((16,), jnp.int32), unroll=8)

        pltpu.sync_copy(w_v, w_hbm.at[pl.ds(pl.multiple_of(b * N, 8), N)])


def _sc_apply(scores_flat, tn_flat):
    mesh = plsc.VectorSubcoreMesh(core_axis_name="c", subcore_axis_name="s")
    f = pl.kernel(
        _sc_apply_body, mesh=mesh,
        compiler_params=pltpu.CompilerParams(needs_layout_passes=False),
        out_type=jax.ShapeDtypeStruct((B * N,), jnp.float32),
        scratch_types=[
            pltpu.VMEM((N,), jnp.float32),
            pltpu.VMEM((32,), jnp.int32),
            pltpu.VMEM((N,), jnp.float32),
        ],
    )
    return f(scores_flat, tn_flat)


@jax.jit
def kernel(z, W1, b1, W2, b2, Wh, bh):
    z2d = z.reshape(B * N, D)

    scores2d = pl.pallas_call(
        _router_body,
        grid=(B * N // NB_A,),
        in_specs=[
            pl.BlockSpec((NB_A, D), lambda i: (i, 0)),
            pl.BlockSpec((D, D), lambda i: (0, 0)),
            pl.BlockSpec((1, D), lambda i: (0, 0)),
            pl.BlockSpec((D, 1), lambda i: (0, 0)),
            pl.BlockSpec((1, 1), lambda i: (0, 0)),
        ],
        out_specs=pl.BlockSpec((NB_A, 1), lambda i: (i, 0)),
        out_shape=jax.ShapeDtypeStruct((B * N, 1), jnp.float32),
    )(z2d, W1, b1.reshape(1, D), W2, b2.reshape(1, 1))

    tn = pl.pallas_call(
        _thresh_body,
        in_specs=[pl.BlockSpec((B, N), lambda: (0, 0))],
        out_specs=pl.BlockSpec((B, 32), lambda: (0, 0)),
        out_shape=jax.ShapeDtypeStruct((B, 32), jnp.int32),
    )(scores2d.reshape(B, N))

    w_flat = _sc_apply(scores2d.reshape(B * N), tn.reshape(B * 32))

    seg2d = pl.pallas_call(
        _seg_body,
        grid=(B * N // NB_SEG,),
        in_specs=[
            pl.BlockSpec((NB_SEG, D), lambda i: (i, 0)),
            pl.BlockSpec((D, C), lambda i: (0, 0)),
            pl.BlockSpec((1, C), lambda i: (0, 0)),
        ],
        out_specs=pl.BlockSpec((NB_SEG, C), lambda i: (i, 0)),
        out_shape=jax.ShapeDtypeStruct((B * N, C), jnp.float32),
    )(z2d, Wh, bh.reshape(1, C))

    clip_logits = pl.pallas_call(
        _clip_body,
        in_specs=[
            pl.BlockSpec((B, N), lambda: (0, 0)),
            pl.BlockSpec((B * N, C), lambda: (0, 0)),
        ],
        out_specs=pl.BlockSpec((B, C), lambda: (0, 0)),
        out_shape=jax.ShapeDtypeStruct((B, C), jnp.float32),
    )(w_flat.reshape(B, N), seg2d)

    return clip_logits, seg2d.reshape(B, N, C), w_flat.reshape(B, N)


# speedup vs baseline: 1.0695x; 1.0695x over previous
import jax
import jax.numpy as jnp
from jax.experimental import pallas as pl
from jax.experimental.pallas import tpu as pltpu
from jax.experimental.pallas import tpu_sc as plsc

B, N, D, C, K = 4, 4096, 1024, 2, 256

NB_A = 1024


def _router_body(z_ref, Wc_ref, b1_ref, W2_ref, b2_ref, bh_ref,
                 scores_ref, seg_ref):
    zb = z_ref[...]
    combined = jax.lax.dot_general(zb, Wc_ref[...], (((1,), (0,)), ((), ())),
                                   preferred_element_type=jnp.float32)
    h = jnp.maximum(combined[:, :D] + b1_ref[...], 0.0)
    seg_ref[...] = combined[:, D:D + C] + bh_ref[...]
    s = jax.lax.dot_general(h, W2_ref[...], (((1,), (0,)), ((), ())),
                            preferred_element_type=jnp.float32) + b2_ref[...]
    scores_ref[...] = s


def _sortable_i32(bits):
    return jnp.where(bits < 0, bits ^ jnp.int32(0x7FFFFFFF), bits)


_NV = N // 16


def _thresh_body(scores_ref, tn_ref):
    kk = _sortable_i32(jax.lax.bitcast_convert_type(scores_ref[...], jnp.int32))
    msb = jnp.int32(-2147483648)

    def step(i, prefix_u):
        cand_u = prefix_u | (jnp.int32(1) << (jnp.int32(31) - i))
        cand_s = cand_u ^ msb
        cnt = jnp.sum((kk >= cand_s).astype(jnp.int32), axis=1, keepdims=True)
        return jnp.where(cnt >= K, cand_u, prefix_u)

    prefix_u = jax.lax.fori_loop(0, 32, step, jnp.zeros((B, 1), jnp.int32))
    t_s = prefix_u ^ msb
    n_gt = jnp.sum((kk > t_s).astype(jnp.int32), axis=1, keepdims=True)
    need = K - n_gt
    tn_ref[...] = jnp.concatenate(
        [jnp.broadcast_to(t_s, (B, 16)), jnp.broadcast_to(need, (B, 16))],
        axis=1)


def _sc_apply_body(scores_hbm, seg_hbm, tn_hbm, w_hbm, clip_hbm,
                   s_v, segp_v, tn_v, w_v, c16_v):
    cid = jax.lax.axis_index("c")
    sid = jax.lax.axis_index("s")
    wid = sid * 2 + cid

    @pl.when(wid < B)
    def _():
        b = wid
        pltpu.sync_copy(scores_hbm.at[pl.ds(pl.multiple_of(b * N, 8), N)], s_v)
        pltpu.sync_copy(
            seg_hbm.at[pl.ds(pl.multiple_of(b * N * C, 8), N * C)], segp_v)
        pltpu.sync_copy(tn_hbm.at[pl.ds(pl.multiple_of(b * 32, 8), 32)], tn_v)

        li = jax.lax.iota(jnp.int32, 16)
        t_vec = tn_v[pl.ds(0, 16)]
        need_v = tn_v[pl.ds(16, 16)]
        inv_k = jnp.float32(1.0 / K)
        zf = jnp.zeros((16,), jnp.float32)
        half = jax.lax.shift_right_logical(li, 1)

        def fin(i, carry):
            ceq_v, acc = carry
            sv = s_v[pl.ds(i * 16, 16)]
            bits = jax.lax.bitcast_convert_type(sv, jnp.int32)
            ks = jnp.where(bits < 0, bits ^ jnp.int32(0x7FFFFFFF), bits)
            gt = ks > t_vec
            eq = ks == t_vec
            ec = plsc.cumsum(eq.astype(jnp.int32))
            sel = jnp.logical_or(
                gt, jnp.logical_and(eq, (ceq_v + ec) <= need_v))
            w_v[pl.ds(i * 16, 16)] = jnp.where(sel, inv_k, 0.0)
            ceq_v = ceq_v + plsc.all_reduce_population_count(eq)
            wexp0 = plsc.load_gather(w_v, [i * 16 + half])
            wexp1 = plsc.load_gather(w_v, [i * 16 + 8 + half])
            acc = acc + wexp0 * segp_v[pl.ds(i * 32, 16)]
            acc = acc + wexp1 * segp_v[pl.ds(i * 32 + 16, 16)]
            return ceq_v, acc
        _, acc = jax.lax.fori_loop(
            0, _NV, fin, (jnp.zeros((16,), jnp.int32), zf), unroll=8)

        even = (li & 1) == 0
        c0 = jnp.sum(jnp.where(even, acc, 0.0))
        c1 = jnp.sum(jnp.where(even, 0.0, acc))
        c16_v[...] = jnp.where(li == 0, c0, jnp.where(li == 1, c1, 0.0))

        pltpu.sync_copy(w_v, w_hbm.at[pl.ds(pl.multiple_of(b * N, 8), N)])
        pltpu.sync_copy(c16_v, clip_hbm.at[pl.ds(pl.multiple_of(b * 16, 8), 16)])


def _sc_apply(scores_flat, seg_flat, tn_flat):
    mesh = plsc.VectorSubcoreMesh(core_axis_name="c", subcore_axis_name="s")
    f = pl.kernel(
        _sc_apply_body, mesh=mesh,
        compiler_params=pltpu.CompilerParams(needs_layout_passes=False),
        out_type=[
            jax.ShapeDtypeStruct((B * N,), jnp.float32),
            jax.ShapeDtypeStruct((B * 16,), jnp.float32),
        ],
        scratch_types=[
            pltpu.VMEM((N,), jnp.float32),
            pltpu.VMEM((N * C,), jnp.float32),
            pltpu.VMEM((32,), jnp.int32),
            pltpu.VMEM((N,), jnp.float32),
            pltpu.VMEM((16,), jnp.float32),
        ],
    )
    return f(scores_flat, seg_flat, tn_flat)


@jax.jit
def kernel(z, W1, b1, W2, b2, Wh, bh):
    z2d = z.reshape(B * N, D)
    Wc = jnp.concatenate([W1, Wh], axis=1)

    scores2d, seg2d = pl.pallas_call(
        _router_body,
        grid=(B * N // NB_A,),
        in_specs=[
            pl.BlockSpec((NB_A, D), lambda i: (i, 0)),
            pl.BlockSpec((D, D + C), lambda i: (0, 0)),
            pl.BlockSpec((1, D), lambda i: (0, 0)),
            pl.BlockSpec((D, 1), lambda i: (0, 0)),
            pl.BlockSpec((1, 1), lambda i: (0, 0)),
            pl.BlockSpec((1, C), lambda i: (0, 0)),
        ],
        out_specs=[
            pl.BlockSpec((NB_A, 1), lambda i: (i, 0)),
            pl.BlockSpec((NB_A, C), lambda i: (i, 0)),
        ],
        out_shape=[
            jax.ShapeDtypeStruct((B * N, 1), jnp.float32),
            jax.ShapeDtypeStruct((B * N, C), jnp.float32),
        ],
    )(z2d, Wc, b1.reshape(1, D), W2, b2.reshape(1, 1), bh.reshape(1, C))

    tn = pl.pallas_call(
        _thresh_body,
        in_specs=[pl.BlockSpec((B, N), lambda: (0, 0))],
        out_specs=pl.BlockSpec((B, 32), lambda: (0, 0)),
        out_shape=jax.ShapeDtypeStruct((B, 32), jnp.int32),
    )(scores2d.reshape(B, N))

    w_flat, clip_pad = _sc_apply(scores2d.reshape(B * N),
                                 seg2d.reshape(B * N * C),
                                 tn.reshape(B * 32))
    clip_logits = clip_pad.reshape(B, 16)[:, :C]
    return clip_logits, seg2d.reshape(B, N, C), w_flat.reshape(B, N)
